# baseline (speedup 1.0000x reference)
import functools

import jax
import jax.numpy as jnp
from jax import lax
from jax.experimental import pallas as pl
from jax.experimental.pallas import tpu as pltpu
from jax.experimental.pallas import tpu_sc as plsc

NUM_CORES = 2
NUM_SUBCORES = 16
NW = NUM_CORES * NUM_SUBCORES
LANES = 16
SEQ = 16384
B_PER_W = SEQ // NW
EMB = 64
NTAGS = 1000
NCHUNK = 4
CHUNK = B_PER_W // NCHUNK


def _sc_gather_sum(nwords_i32, emb_weight):
    mesh = plsc.VectorSubcoreMesh(core_axis_name="c", subcore_axis_name="s")

    @functools.partial(
        pl.kernel,
        out_type=jax.ShapeDtypeStruct((NW, EMB), jnp.float32),
        mesh=mesh,
        scratch_types=[
            pltpu.VMEM((B_PER_W,), jnp.int32),
            pltpu.VMEM((B_PER_W, EMB), jnp.float32),
            pltpu.VMEM((EMB,), jnp.float32),
            pltpu.SemaphoreType.DMA,
        ],
        compiler_params=pltpu.CompilerParams(use_tc_tiling_on_sc=False),
    )
    def k(idx_hbm, table_hbm, out_hbm, idx_v, rows_v, acc_v, sem):
        wid = lax.axis_index("s") * NUM_CORES + lax.axis_index("c")
        base = wid * B_PER_W
        pltpu.sync_copy(idx_hbm.at[pl.ds(base, B_PER_W)], idx_v)
        copies = [
            pltpu.async_copy(
                table_hbm.at[idx_v.at[pl.ds(j * CHUNK, CHUNK)]],
                rows_v.at[pl.ds(j * CHUNK, CHUNK)],
                sem,
            )
            for j in range(NCHUNK)
        ]
        for cp in copies:
            cp.wait()

        def body(i, accs):
            return tuple(
                accs[e] + rows_v[i, pl.ds(e * LANES, LANES)] for e in range(4)
            )

        accs = lax.fori_loop(
            0, B_PER_W, body,
            tuple(jnp.zeros((LANES,), jnp.float32) for _ in range(4)),
        )
        for e in range(4):
            acc_v[pl.ds(e * LANES, LANES)] = accs[e]
        pltpu.sync_copy(acc_v, out_hbm.at[wid])

    return k(nwords_i32, emb_weight)


def _tc_head(partials, lin_weight, lin_bias2d):
    def body(p_ref, w_ref, b_ref, o_ref):
        s = jnp.sum(p_ref[...], axis=0, keepdims=True)
        o_ref[...] = (
            lax.dot_general(
                s, w_ref[...],
                dimension_numbers=(((1,), (1,)), ((), ())),
                preferred_element_type=jnp.float32,
            )
            + b_ref[...]
        )

    return pl.pallas_call(
        body,
        out_shape=jax.ShapeDtypeStruct((1, NTAGS), jnp.float32),
    )(partials, lin_weight, lin_bias2d)


def kernel(nwords, emb_weight, lin_weight, lin_bias):
    idx = nwords.astype(jnp.int32)
    partials = _sc_gather_sum(idx, emb_weight)
    return _tc_head(partials, lin_weight, lin_bias.reshape(1, NTAGS))
